# Initial kernel scaffold; baseline (speedup 1.0000x reference)
import jax
import jax.numpy as jnp
from jax.experimental import pallas as pl


def kernel(x):
    raise NotImplementedError("write your pallas kernel here")


# speedup vs baseline: 3.2618x; 3.2618x over previous
import functools

import jax
import jax.numpy as jnp
from jax import lax
from jax.experimental import pallas as pl
from jax.experimental.pallas import tpu as pltpu
from jax.experimental.pallas import tpu_sc as plsc

TOKENS = 98

NW = 32
CHUNK = 72


def _selected_token_indices(N, L):
    noise = jax.random.uniform(jax.random.key(1), (N, L - 1), dtype=jnp.float32)
    patch_mask = jnp.argsort(noise, axis=1) + 1
    patch_mask = patch_mask[:, :TOKENS]
    patch_mask = jnp.sort(patch_mask, axis=1)
    cls_mask = jnp.zeros((N, 1), dtype=patch_mask.dtype)
    return jnp.concatenate([cls_mask, patch_mask], axis=1)


def _make_gather(B, D, nchunks):
    mesh = plsc.VectorSubcoreMesh(core_axis_name="c", subcore_axis_name="s")
    b_per_w = B // NW

    @functools.partial(
        pl.kernel,
        mesh=mesh,
        out_type=jax.ShapeDtypeStruct((B, D), jnp.float32),
        scratch_types=[
            pltpu.VMEM((nchunks, CHUNK), jnp.int32),
            pltpu.VMEM((CHUNK, D), jnp.float32),
            pltpu.SemaphoreType.DMA,
        ],
    )
    def gather_rows(table_hbm, idx_hbm, out_hbm, idx_v, rows_v, sem):
        wid = lax.axis_index("s") * 2 + lax.axis_index("c")
        base = wid * b_per_w
        pltpu.sync_copy(idx_hbm.at[wid], idx_v)
        for j in range(nchunks):
            pltpu.async_copy(table_hbm.at[idx_v.at[j]], rows_v, sem).wait()
            pltpu.sync_copy(rows_v, out_hbm.at[pl.ds(base + j * CHUNK, CHUNK)])

    return gather_rows


def kernel(x):
    N, L, D = x.shape
    mask = _selected_token_indices(N, L)
    T = TOKENS + 1
    B = N * T
    flat_idx = (jnp.arange(N, dtype=mask.dtype)[:, None] * L + mask).reshape(-1)
    nchunks = B // (NW * CHUNK)
    idx3 = flat_idx.reshape(NW, nchunks, CHUNK).astype(jnp.int32)
    table = x.reshape(N * L, D)
    out = _make_gather(B, D, nchunks)(table, idx3)
    return out.reshape(N, T, D)


# speedup vs baseline: 3.3152x; 1.0164x over previous
import functools

import jax
import jax.numpy as jnp
from jax import lax
from jax.experimental import pallas as pl
from jax.experimental.pallas import tpu as pltpu
from jax.experimental.pallas import tpu_sc as plsc

TOKENS = 98

NW = 32
CHUNK = 72


def _selected_token_indices(N, L):
    noise = jax.random.uniform(jax.random.key(1), (N, L - 1), dtype=jnp.float32)
    patch_mask = jnp.argsort(noise, axis=1) + 1
    patch_mask = patch_mask[:, :TOKENS]
    patch_mask = jnp.sort(patch_mask, axis=1)
    cls_mask = jnp.zeros((N, 1), dtype=patch_mask.dtype)
    return jnp.concatenate([cls_mask, patch_mask], axis=1)


def _make_gather(B, D, nchunks):
    mesh = plsc.VectorSubcoreMesh(core_axis_name="c", subcore_axis_name="s")
    b_per_w = B // NW

    @functools.partial(
        pl.kernel,
        mesh=mesh,
        out_type=jax.ShapeDtypeStruct((B, D), jnp.float32),
        scratch_types=[
            pltpu.VMEM((nchunks, CHUNK), jnp.int32),
            pltpu.VMEM((CHUNK, D), jnp.float32),
            pltpu.VMEM((CHUNK, D), jnp.float32),
            pltpu.SemaphoreType.DMA,
            pltpu.SemaphoreType.DMA,
            pltpu.SemaphoreType.DMA,
            pltpu.SemaphoreType.DMA,
        ],
    )
    def gather_rows(table_hbm, idx_hbm, out_hbm, idx_v,
                    buf0, buf1, g0, g1, s0, s1):
        wid = lax.axis_index("s") * 2 + lax.axis_index("c")
        base = wid * b_per_w
        bufs, gsems, ssems = (buf0, buf1), (g0, g1), (s0, s1)
        pltpu.sync_copy(idx_hbm.at[wid], idx_v)
        gh = [None, None]
        sh = [None, None]
        gh[0] = pltpu.async_copy(table_hbm.at[idx_v.at[0]], bufs[0], gsems[0])
        for j in range(nchunks):
            p, q = j % 2, (j + 1) % 2
            if j + 1 < nchunks:
                if sh[q] is not None:
                    sh[q].wait()
                gh[q] = pltpu.async_copy(
                    table_hbm.at[idx_v.at[j + 1]], bufs[q], gsems[q])
            gh[p].wait()
            sh[p] = pltpu.async_copy(
                bufs[p], out_hbm.at[pl.ds(base + j * CHUNK, CHUNK)], ssems[p])
        sh[(nchunks - 1) % 2].wait()
        if nchunks > 1:
            sh[(nchunks - 2) % 2].wait()

    return gather_rows


def kernel(x):
    N, L, D = x.shape
    mask = _selected_token_indices(N, L)
    T = TOKENS + 1
    B = N * T
    flat_idx = (jnp.arange(N, dtype=mask.dtype)[:, None] * L + mask).reshape(-1)
    nchunks = B // (NW * CHUNK)
    idx3 = flat_idx.reshape(NW, nchunks, CHUNK).astype(jnp.int32)
    table = x.reshape(N * L, D)
    out = _make_gather(B, D, nchunks)(table, idx3)
    return out.reshape(N, T, D)


# speedup vs baseline: 17.0762x; 5.1508x over previous
import functools

import jax
import jax.numpy as jnp
from jax import lax
from jax.experimental import pallas as pl
from jax.experimental.pallas import tpu as pltpu
from jax.experimental.pallas import tpu_sc as plsc

TOKENS = 98

NW = 32


def _selected_token_indices(N, L):
    noise = jax.random.uniform(jax.random.key(1), (N, L - 1), dtype=jnp.float32)
    patch_mask = jnp.argsort(noise, axis=1) + 1
    patch_mask = patch_mask[:, :TOKENS]
    patch_mask = jnp.sort(patch_mask, axis=1)
    cls_mask = jnp.zeros((N, 1), dtype=patch_mask.dtype)
    return jnp.concatenate([cls_mask, patch_mask], axis=1)


CHUNK = 72


def _make_gather(B, D, nchunks):
    mesh = plsc.VectorSubcoreMesh(core_axis_name="c", subcore_axis_name="s")
    b_per_w = B // NW

    @functools.partial(
        pl.kernel,
        mesh=mesh,
        out_type=jax.ShapeDtypeStruct((B, D), jnp.float32),
        scratch_types=[
            pltpu.VMEM((nchunks, CHUNK), jnp.int32),
            pltpu.VMEM((CHUNK, D), jnp.float32),
            pltpu.SemaphoreType.DMA,
        ],
    )
    def gather_rows(table_hbm, idx_hbm, out_hbm, idx_v, buf, sem):
        wid = lax.axis_index("s") * 2 + lax.axis_index("c")
        base = wid * b_per_w
        pltpu.sync_copy(idx_hbm.at[wid], idx_v)
        for j in range(nchunks):
            pltpu.async_copy(table_hbm.at[idx_v.at[j]], buf, sem).wait()
            pltpu.sync_copy(buf, out_hbm.at[pl.ds(base + j * CHUNK, CHUNK)])

    return gather_rows


def kernel(x):
    N, L, D = x.shape
    T = TOKENS + 1
    mask = _selected_token_indices(N, L)
    B = N * T
    src = mask.T * N + jnp.arange(N, dtype=mask.dtype)[None, :]
    nchunks = B // (NW * CHUNK)
    idx3 = src.reshape(NW, nchunks, CHUNK).astype(jnp.int32)
    xt = jnp.transpose(x, (1, 0, 2)).reshape(L * N, D)
    out2 = _make_gather(B, D, nchunks)(xt, idx3)
    return jnp.transpose(out2.reshape(T, N, D), (1, 0, 2))


# speedup vs baseline: 20.2271x; 1.1845x over previous
import functools

import jax
import jax.numpy as jnp
import numpy as np
from jax import lax
from jax.experimental import pallas as pl
from jax.experimental.pallas import tpu as pltpu
from jax.experimental.pallas import tpu_sc as plsc

TOKENS = 98

NW = 32
CHUNK = 72


def _uniform_threefry_np(seed, rows, cols):
    size = rows * cols
    i = np.arange(size, dtype=np.uint64)
    x = [
        (i >> np.uint64(32)).astype(np.uint32),
        (i & np.uint64(0xFFFFFFFF)).astype(np.uint32),
    ]
    k0 = np.uint32(seed >> 32)
    k1 = np.uint32(seed & 0xFFFFFFFF)
    ks = [k0, k1, np.uint32(k0 ^ k1 ^ np.uint32(0x1BD11BDA))]
    rotations = [
        np.array([13, 15, 26, 6], dtype=np.uint32),
        np.array([17, 29, 16, 24], dtype=np.uint32),
    ]

    def rotl(v, d):
        return (v << d) | (v >> np.uint32(32 - int(d)))

    x[0] = x[0] + ks[0]
    x[1] = x[1] + ks[1]
    for r5 in range(5):
        for r in rotations[r5 % 2]:
            x[0] = x[0] + x[1]
            x[1] = x[0] ^ rotl(x[1], r)
        x[0] = x[0] + ks[(r5 + 1) % 3]
        x[1] = x[1] + ks[(r5 + 2) % 3] + np.uint32(r5 + 1)
    bits = (x[0] ^ x[1]).reshape(rows, cols)
    float_bits = (bits >> np.uint32(9)) | np.uint32(0x3F800000)
    floats = float_bits.view(np.float32) - np.float32(1.0)
    return np.maximum(np.float32(0.0), floats)


def _selected_token_indices(N, L):
    noise = _uniform_threefry_np(1, N, L - 1)
    patch_mask = np.argsort(noise, axis=1, kind="stable") + 1
    patch_mask = np.sort(patch_mask[:, :TOKENS], axis=1)
    cls_mask = np.zeros((N, 1), dtype=patch_mask.dtype)
    return np.concatenate([cls_mask, patch_mask], axis=1)


def _make_gather(B, D, nchunks):
    mesh = plsc.VectorSubcoreMesh(core_axis_name="c", subcore_axis_name="s")
    b_per_w = B // NW

    @functools.partial(
        pl.kernel,
        mesh=mesh,
        out_type=jax.ShapeDtypeStruct((B, D), jnp.float32),
        scratch_types=[
            pltpu.VMEM((nchunks, CHUNK), jnp.int32),
            pltpu.VMEM((CHUNK, D), jnp.float32),
            pltpu.VMEM((CHUNK, D), jnp.float32),
            pltpu.SemaphoreType.DMA,
            pltpu.SemaphoreType.DMA,
            pltpu.SemaphoreType.DMA,
            pltpu.SemaphoreType.DMA,
        ],
    )
    def gather_rows(table_hbm, idx_hbm, out_hbm, idx_v,
                    buf0, buf1, g0, g1, s0, s1):
        wid = lax.axis_index("s") * 2 + lax.axis_index("c")
        base = wid * b_per_w
        bufs, gsems, ssems = (buf0, buf1), (g0, g1), (s0, s1)
        pltpu.sync_copy(idx_hbm.at[wid], idx_v)
        gh = [None, None]
        sh = [None, None]
        gh[0] = pltpu.async_copy(table_hbm.at[idx_v.at[0]], bufs[0], gsems[0])
        for j in range(nchunks):
            p, q = j % 2, (j + 1) % 2
            if j + 1 < nchunks:
                if sh[q] is not None:
                    sh[q].wait()
                gh[q] = pltpu.async_copy(
                    table_hbm.at[idx_v.at[j + 1]], bufs[q], gsems[q])
            gh[p].wait()
            sh[p] = pltpu.async_copy(
                bufs[p], out_hbm.at[pl.ds(base + j * CHUNK, CHUNK)], ssems[p])
        sh[(nchunks - 1) % 2].wait()
        if nchunks > 1:
            sh[(nchunks - 2) % 2].wait()

    return gather_rows


def kernel(x):
    N, L, D = x.shape
    T = TOKENS + 1
    mask = _selected_token_indices(N, L)
    B = N * T
    src = mask.T.astype(np.int64) * N + np.arange(N, dtype=np.int64)[None, :]
    nchunks = B // (NW * CHUNK)
    idx3 = jnp.asarray(src.reshape(NW, nchunks, CHUNK).astype(np.int32))
    xt = jnp.transpose(x, (1, 0, 2)).reshape(L * N, D)
    out2 = _make_gather(B, D, nchunks)(xt, idx3)
    return jnp.transpose(out2.reshape(T, N, D), (1, 0, 2))
